# Initial kernel scaffold; baseline (speedup 1.0000x reference)
import jax
import jax.numpy as jnp
from jax.experimental import pallas as pl


def kernel(feature, edge_index, embedding, W_conv, W_mlp):
    raise NotImplementedError("write your pallas kernel here")


# speedup vs baseline: 22.4813x; 22.4813x over previous
import functools

import jax
import jax.numpy as jnp
from jax import lax
from jax.experimental import pallas as pl
from jax.experimental.pallas import tpu as pltpu
from jax.experimental.pallas import tpu_sc as plsc

N = 10000
E = 320000
D = 128

NC = 2
NS = 16
NW = NC * NS
EPW = E // NW
CHUNK = 80
NCHUNK = EPW // CHUNK
DEG_PT = 632
DEG_PAD = DEG_PT * NS

H = 5120
TRASH = H
ACC_PT = 328
ACC_R = ACC_PT * NS
EPT = E // NS
MCHUNK = EPT // CHUNK

_mesh = plsc.VectorSubcoreMesh(core_axis_name="c", subcore_axis_name="s")


def _deg_body(dst_hbm, zeros_hbm, ones_hbm, deg_out, idx_v, ones_v, hist_sh,
              sem):
    c = lax.axis_index("c")
    s = lax.axis_index("s")
    w = c * NS + s
    pltpu.sync_copy(dst_hbm.at[w], idx_v)
    pltpu.sync_copy(ones_hbm, ones_v)
    pltpu.sync_copy(zeros_hbm, hist_sh.at[pl.ds(s * DEG_PT, DEG_PT)])
    plsc.subcore_barrier()

    def chunk(t, carry):
        pltpu.sync_copy(ones_v, hist_sh.at[idx_v.at[t]], add=True)
        return carry

    lax.fori_loop(0, NCHUNK, chunk, 0)
    plsc.subcore_barrier()
    pltpu.sync_copy(hist_sh.at[pl.ds(s * DEG_PT, DEG_PT)],
                    deg_out.at[c, pl.ds(s * DEG_PT, DEG_PT)])


_deg_call = pl.kernel(
    _deg_body,
    out_type=jax.ShapeDtypeStruct((NC, DEG_PAD, 1), jnp.float32),
    mesh=_mesh,
    scratch_types=[
        pltpu.VMEM((NCHUNK, CHUNK), jnp.int32),
        pltpu.VMEM((CHUNK, 1), jnp.float32),
        pltpu.VMEM_SHARED((DEG_PAD, 1), jnp.float32),
        pltpu.SemaphoreType.DMA,
    ],
)


def _msg_body(y_hbm, src_hbm, dstl_hbm, zrows_hbm, acc_out, srcv, dstv,
              rows_a, rows_b, acc_sh, sem_a, sem_b):
    c = lax.axis_index("c")
    s = lax.axis_index("s")
    pltpu.sync_copy(src_hbm.at[s], srcv)
    pltpu.sync_copy(dstl_hbm.at[c, s], dstv)
    pltpu.sync_copy(zrows_hbm, acc_sh.at[pl.ds(s * ACC_PT, ACC_PT)])
    plsc.subcore_barrier()

    pltpu.async_copy(y_hbm.at[srcv.at[0]], rows_a, sem_a)

    def pair(t, carry):
        j = 2 * t
        pltpu.async_copy(y_hbm.at[srcv.at[j + 1]], rows_b, sem_b)
        pltpu.make_async_copy(y_hbm.at[srcv.at[j]], rows_a, sem_a).wait()
        pltpu.sync_copy(rows_a, acc_sh.at[dstv.at[j]], add=True)

        @pl.when(j + 2 < MCHUNK)
        def _():
            pltpu.async_copy(y_hbm.at[srcv.at[j + 2]], rows_a, sem_a)

        pltpu.make_async_copy(y_hbm.at[srcv.at[j + 1]], rows_b, sem_b).wait()
        pltpu.sync_copy(rows_b, acc_sh.at[dstv.at[j + 1]], add=True)
        return carry

    lax.fori_loop(0, MCHUNK // 2, pair, 0)
    plsc.subcore_barrier()
    pltpu.sync_copy(acc_sh.at[pl.ds(s * ACC_PT, ACC_PT)],
                    acc_out.at[c, pl.ds(s * ACC_PT, ACC_PT)])


_msg_call = pl.kernel(
    _msg_body,
    out_type=jax.ShapeDtypeStruct((NC, ACC_R, D), jnp.float32),
    mesh=_mesh,
    scratch_types=[
        pltpu.VMEM((MCHUNK, CHUNK), jnp.int32),
        pltpu.VMEM((MCHUNK, CHUNK), jnp.int32),
        pltpu.VMEM((CHUNK, D), jnp.float32),
        pltpu.VMEM((CHUNK, D), jnp.float32),
        pltpu.VMEM_SHARED((ACC_R, D), jnp.float32),
        pltpu.SemaphoreType.DMA,
        pltpu.SemaphoreType.DMA,
    ],
)


def _scale_body(emb_ref, wct_ref, degp_ref, y_ref, dis_ref):
    degp = degp_ref[...]
    deg = 1.0 + degp[0, :N] + degp[1, :N]
    dis = lax.rsqrt(deg)
    xw = jnp.dot(emb_ref[...], wct_ref[...],
                 preferred_element_type=jnp.float32,
                 precision=lax.Precision.HIGHEST)
    y_ref[...] = dis * xw
    dis_ref[...] = dis


def _combine_body(acc_ref, y_ref, dis_ref, feat_ref, wmt_ref, out_ref):
    acc = acc_ref[...]
    accfull = jnp.concatenate([acc[0, :H], acc[1, :N - H]], axis=0)
    mlp = jnp.dot(feat_ref[...], wmt_ref[...],
                  preferred_element_type=jnp.float32,
                  precision=lax.Precision.HIGHEST)
    gcn = dis_ref[...] * (accfull + y_ref[...])
    out_ref[...] = jnp.maximum(gcn + mlp, 0.0)


def kernel(feature, edge_index, embedding, W_conv, W_mlp):
    src = edge_index[0].astype(jnp.int32)
    dst = edge_index[1].astype(jnp.int32)
    src_deg = src.reshape(NW, NCHUNK, CHUNK)
    dst_deg = dst.reshape(NW, NCHUNK, CHUNK)
    halves = []
    for c in range(NC):
        local = dst - c * H
        halves.append(jnp.where((local >= 0) & (local < H), local, TRASH))
    dstl = jnp.stack(halves).reshape(NC, NS, MCHUNK, CHUNK)
    src_msg = src.reshape(NS, MCHUNK, CHUNK)

    zeros_deg = jnp.zeros((DEG_PT, 1), jnp.float32)
    ones_col = jnp.ones((CHUNK, 1), jnp.float32)
    degp = _deg_call(dst_deg, zeros_deg, ones_col)

    y, dis = pl.pallas_call(
        _scale_body,
        out_shape=(
            jax.ShapeDtypeStruct((N, D), jnp.float32),
            jax.ShapeDtypeStruct((N, 1), jnp.float32),
        ),
    )(embedding, W_conv.T, degp)

    zrows = jnp.zeros((ACC_PT, D), jnp.float32)
    acc = _msg_call(y, src_msg, dstl, zrows)

    out = pl.pallas_call(
        _combine_body,
        out_shape=jax.ShapeDtypeStruct((N, D), jnp.float32),
    )(acc, y, dis, feature, W_mlp.T)
    return out


# speedup vs baseline: 22.5746x; 1.0042x over previous
import functools

import jax
import jax.numpy as jnp
from jax import lax
from jax.experimental import pallas as pl
from jax.experimental.pallas import tpu as pltpu
from jax.experimental.pallas import tpu_sc as plsc

N = 10000
E = 320000
D = 128

NC = 2
NS = 16
NW = NC * NS
EPW = E // NW
CHUNK = 80
NCHUNK = EPW // CHUNK
NBUF = 2
DEG_PT = 632
DEG_PAD = DEG_PT * NS

H = 5000
TRASH = H
ACC_PT = 320
ACC_R = ACC_PT * NS
EPT = E // NS
MCHUNK = EPT // CHUNK
M_MAIN = (MCHUNK // NBUF) * NBUF
M_REM = MCHUNK - M_MAIN

_mesh = plsc.VectorSubcoreMesh(core_axis_name="c", subcore_axis_name="s")


def _deg_body(dst_hbm, zeros_hbm, ones_hbm, deg_out, idx_v, ones_v, hist_sh,
              sem):
    c = lax.axis_index("c")
    s = lax.axis_index("s")
    w = c * NS + s
    pltpu.sync_copy(dst_hbm.at[w], idx_v)
    pltpu.sync_copy(ones_hbm, ones_v)
    pltpu.sync_copy(zeros_hbm, hist_sh.at[pl.ds(s * DEG_PT, DEG_PT)])
    plsc.subcore_barrier()

    def chunk(t, carry):
        pltpu.sync_copy(ones_v, hist_sh.at[idx_v.at[t]], add=True)
        return carry

    lax.fori_loop(0, NCHUNK, chunk, 0)
    plsc.subcore_barrier()
    pltpu.sync_copy(hist_sh.at[pl.ds(s * DEG_PT, DEG_PT)],
                    deg_out.at[c, pl.ds(s * DEG_PT, DEG_PT)])


_deg_call = pl.kernel(
    _deg_body,
    out_type=jax.ShapeDtypeStruct((NC, DEG_PAD, 1), jnp.float32),
    mesh=_mesh,
    scratch_types=[
        pltpu.VMEM((NCHUNK, CHUNK), jnp.int32),
        pltpu.VMEM((CHUNK, 1), jnp.float32),
        pltpu.VMEM_SHARED((DEG_PAD, 1), jnp.float32),
        pltpu.SemaphoreType.DMA,
    ],
)


def _msg_body(y_hbm, src_hbm, dstl_hbm, zrows_hbm, acc_out, srcv, dstv,
              rows, gsem, ssem, acc_sh):
    c = lax.axis_index("c")
    s = lax.axis_index("s")
    pltpu.sync_copy(src_hbm.at[s], srcv)
    pltpu.sync_copy(dstl_hbm.at[c, s], dstv)
    pltpu.sync_copy(zrows_hbm, acc_sh.at[pl.ds(s * ACC_PT, ACC_PT)])
    plsc.subcore_barrier()

    for b in range(NBUF):
        pltpu.async_copy(y_hbm.at[srcv.at[b]], rows[b], gsem[b])

    def quad(t, carry):
        j0 = NBUF * t
        for b in range(NBUF):
            j = j0 + b
            pltpu.make_async_copy(y_hbm.at[srcv.at[j]], rows[b],
                                  gsem[b]).wait()
            pltpu.sync_copy(rows[b], acc_sh.at[dstv.at[j]], add=True)

            @pl.when(j + NBUF < MCHUNK)
            def _():
                pltpu.async_copy(y_hbm.at[srcv.at[j + NBUF]], rows[b],
                                 gsem[b])

        return carry

    lax.fori_loop(0, M_MAIN // NBUF, quad, 0)
    for r in range(M_REM):
        j = M_MAIN + r
        pltpu.make_async_copy(y_hbm.at[srcv.at[j]], rows[r], gsem[r]).wait()
        pltpu.sync_copy(rows[r], acc_sh.at[dstv.at[j]], add=True)
    plsc.subcore_barrier()
    pltpu.sync_copy(acc_sh.at[pl.ds(s * ACC_PT, ACC_PT)],
                    acc_out.at[c, pl.ds(s * ACC_PT, ACC_PT)])


_msg_call = pl.kernel(
    _msg_body,
    out_type=jax.ShapeDtypeStruct((NC, ACC_R, D), jnp.float32),
    mesh=_mesh,
    scratch_types=[
        pltpu.VMEM((MCHUNK, CHUNK), jnp.int32),
        pltpu.VMEM((MCHUNK, CHUNK), jnp.int32),
        [pltpu.VMEM((CHUNK, D), jnp.float32) for _ in range(NBUF)],
        [pltpu.SemaphoreType.DMA for _ in range(NBUF)],
        [pltpu.SemaphoreType.DMA for _ in range(NBUF)],
        pltpu.VMEM_SHARED((ACC_R, D), jnp.float32),
    ],
)


def _scale_body(emb_ref, wct_ref, degp_ref, y_ref, dis_ref):
    degp = degp_ref[...]
    deg = 1.0 + degp[0, :N] + degp[1, :N]
    dis = lax.rsqrt(deg)
    xw = jnp.dot(emb_ref[...], wct_ref[...],
                 preferred_element_type=jnp.float32,
                 precision=lax.Precision.HIGHEST)
    y_ref[...] = dis * xw
    dis_ref[...] = dis


def _combine_body(acc_ref, y_ref, dis_ref, feat_ref, wmt_ref, out_ref):
    acc = acc_ref[...]
    accfull = jnp.concatenate([acc[0, :H], acc[1, :N - H]], axis=0)
    mlp = jnp.dot(feat_ref[...], wmt_ref[...],
                  preferred_element_type=jnp.float32,
                  precision=lax.Precision.HIGHEST)
    gcn = dis_ref[...] * (accfull + y_ref[...])
    out_ref[...] = jnp.maximum(gcn + mlp, 0.0)


def kernel(feature, edge_index, embedding, W_conv, W_mlp):
    src = edge_index[0].astype(jnp.int32)
    dst = edge_index[1].astype(jnp.int32)
    src_deg = src.reshape(NW, NCHUNK, CHUNK)
    dst_deg = dst.reshape(NW, NCHUNK, CHUNK)
    halves = []
    for c in range(NC):
        local = dst - c * H
        halves.append(jnp.where((local >= 0) & (local < H), local, TRASH))
    dstl = jnp.stack(halves).reshape(NC, NS, MCHUNK, CHUNK)
    src_msg = src.reshape(NS, MCHUNK, CHUNK)

    zeros_deg = jnp.zeros((DEG_PT, 1), jnp.float32)
    ones_col = jnp.ones((CHUNK, 1), jnp.float32)
    degp = _deg_call(dst_deg, zeros_deg, ones_col)

    y, dis = pl.pallas_call(
        _scale_body,
        out_shape=(
            jax.ShapeDtypeStruct((N, D), jnp.float32),
            jax.ShapeDtypeStruct((N, 1), jnp.float32),
        ),
    )(embedding, W_conv.T, degp)

    zrows = jnp.zeros((ACC_PT, D), jnp.float32)
    acc = _msg_call(y, src_msg, dstl, zrows)

    out = pl.pallas_call(
        _combine_body,
        out_shape=jax.ShapeDtypeStruct((N, D), jnp.float32),
    )(acc, y, dis, feature, W_mlp.T)
    return out
